# baseline (speedup 1.0000x reference)
import functools

import jax
import jax.numpy as jnp
from jax import lax
from jax.experimental import pallas as pl
from jax.experimental.pallas import tpu as pltpu
from jax.experimental.pallas import tpu_sc as plsc

TOPK = 8
BLOCK_M = 1024
NC = 2
NS = 16
NW = NC * NS
LANES = 16


def _gate_body(scale_ref, x_ref, w_ref, lt_out):
    lt = lax.dot_general(
        w_ref[...], x_ref[...], (((1,), (1,)), ((), ())),
        preferred_element_type=jnp.float32)
    lt = lt * scale_ref[0]
    tpw = lt_out.shape[2]
    for j in range(lt_out.shape[0]):
        lt_out[j] = lt[:, j * tpw:(j + 1) * tpw]


def _sc_topk_kernel(lt_hbm, w_hbm, i_hbm, lt_v, w_v, i_v):
    n_exp = lt_v.shape[0]
    tpw = lt_v.shape[1]
    wid = lax.axis_index("s") * NC + lax.axis_index("c")
    pltpu.sync_copy(lt_hbm.at[wid], lt_v)

    def group_body(g, _):
        base = g * LANES

        def insert_body(e, carry):
            vals, idxs = carry
            cur_v = lt_v[e, pl.ds(base, LANES)]
            cur_i = jnp.full((LANES,), e, jnp.int32)
            new_v, new_i = [], []
            for k in range(TOPK):
                gt = cur_v > vals[k]
                new_v.append(jnp.where(gt, cur_v, vals[k]))
                new_i.append(jnp.where(gt, cur_i, idxs[k]))
                cur_v = jnp.where(gt, vals[k], cur_v)
                cur_i = jnp.where(gt, idxs[k], cur_i)
            return tuple(new_v), tuple(new_i)

        init = (tuple(jnp.full((LANES,), -jnp.inf, jnp.float32)
                      for _ in range(TOPK)),
                tuple(jnp.zeros((LANES,), jnp.int32) for _ in range(TOPK)))
        vals, idxs = lax.fori_loop(0, n_exp, insert_body, init)
        m = vals[0]
        exps = [jnp.exp(vals[k] - m) for k in range(TOPK)]
        tot = exps[0]
        for k in range(1, TOPK):
            tot = tot + exps[k]
        for k in range(TOPK):
            w_v[k, pl.ds(base, LANES)] = exps[k] / tot
            i_v[k, pl.ds(base, LANES)] = idxs[k]
        return 0

    lax.fori_loop(0, tpw // LANES, group_body, 0)
    pltpu.sync_copy(w_v, w_hbm.at[wid])
    pltpu.sync_copy(i_v, i_hbm.at[wid])


@jax.jit
def kernel(x, W, router_scale):
    tokens, dim = x.shape
    n_exp = W.shape[0]
    tpw = tokens // NW
    wpb = BLOCK_M // tpw
    grid = (tokens // BLOCK_M,)
    lt_tiles = pl.pallas_call(
        _gate_body,
        grid_spec=pltpu.PrefetchScalarGridSpec(
            num_scalar_prefetch=1,
            grid=grid,
            in_specs=[
                pl.BlockSpec((BLOCK_M, dim), lambda i, s: (i, 0)),
                pl.BlockSpec((n_exp, dim), lambda i, s: (0, 0)),
            ],
            out_specs=[
                pl.BlockSpec((wpb, n_exp, tpw), lambda i, s: (i, 0, 0)),
            ],
        ),
        out_shape=[jax.ShapeDtypeStruct((NW, n_exp, tpw), jnp.float32)],
        compiler_params=pltpu.CompilerParams(
            dimension_semantics=("arbitrary",),
        ),
    )(router_scale, x, W)[0]

    mesh = plsc.VectorSubcoreMesh(core_axis_name="c", subcore_axis_name="s")
    wt3, it3 = functools.partial(
        pl.kernel,
        mesh=mesh,
        out_type=[
            jax.ShapeDtypeStruct((NW, TOPK, tpw), jnp.float32),
            jax.ShapeDtypeStruct((NW, TOPK, tpw), jnp.int32),
        ],
        scratch_types=[
            pltpu.VMEM((n_exp, tpw), jnp.float32),
            pltpu.VMEM((TOPK, tpw), jnp.float32),
            pltpu.VMEM((TOPK, tpw), jnp.int32),
        ],
    )(_sc_topk_kernel)(lt_tiles)

    weights = wt3.transpose(0, 2, 1).reshape(tokens, TOPK)
    indices = it3.transpose(0, 2, 1).reshape(tokens, TOPK)
    return (weights, indices)


# speedup vs baseline: 1.1272x; 1.1272x over previous
import functools

import jax
import jax.numpy as jnp
from jax.experimental import pallas as pl
from jax.experimental.pallas import tpu as pltpu

TOPK = 8
BLOCK_M = 512


def _router_body(scale_ref, x_ref, w_ref, w_out, i_out):
    lt = jax.lax.dot_general(
        w_ref[...], x_ref[...], (((1,), (1,)), ((), ())),
        preferred_element_type=jnp.float32)
    cur = lt * scale_ref[0]
    n_exp = cur.shape[0]
    row = jax.lax.broadcasted_iota(jnp.int32, cur.shape, 0)
    vals, idxs = [], []
    for _ in range(TOPK):
        m = jnp.max(cur, axis=0, keepdims=True)
        idx = jnp.min(jnp.where(cur == m, row, n_exp), axis=0, keepdims=True)
        vals.append(m)
        idxs.append(idx)
        cur = jnp.where(row == idx, -jnp.inf, cur)
    w = jnp.concatenate(vals, axis=0)
    e = jnp.exp(w - w[:1])
    w = e / jnp.sum(e, axis=0, keepdims=True)
    w_out[...] = w.T
    i_out[...] = jnp.concatenate(idxs, axis=0).T


@jax.jit
def kernel(x, W, router_scale):
    tokens, dim = x.shape
    n_exp = W.shape[0]
    grid = (tokens // BLOCK_M,)
    weights, indices = pl.pallas_call(
        _router_body,
        grid_spec=pltpu.PrefetchScalarGridSpec(
            num_scalar_prefetch=1,
            grid=grid,
            in_specs=[
                pl.BlockSpec((BLOCK_M, dim), lambda i, s: (i, 0)),
                pl.BlockSpec((n_exp, dim), lambda i, s: (0, 0)),
            ],
            out_specs=[
                pl.BlockSpec((BLOCK_M, TOPK), lambda i, s: (i, 0)),
                pl.BlockSpec((BLOCK_M, TOPK), lambda i, s: (i, 0)),
            ],
        ),
        out_shape=[
            jax.ShapeDtypeStruct((tokens, TOPK), jnp.float32),
            jax.ShapeDtypeStruct((tokens, TOPK), jnp.int32),
        ],
        compiler_params=pltpu.CompilerParams(
            dimension_semantics=("arbitrary",),
            vmem_limit_bytes=128 * 1024 * 1024,
        ),
    )(router_scale, x, W)
    return (weights, indices)


# speedup vs baseline: 1.1960x; 1.0611x over previous
import functools

import jax
import jax.numpy as jnp
from jax.experimental import pallas as pl
from jax.experimental.pallas import tpu as pltpu

TOPK = 8
BLOCK_M = 1024


def _router_body(scale_ref, x_ref, w_ref, w_out, i_out):
    lt = jax.lax.dot_general(
        w_ref[...], x_ref[...], (((1,), (1,)), ((), ())),
        preferred_element_type=jnp.float32)
    cur = lt * scale_ref[0]
    n_exp = cur.shape[0]
    row = jax.lax.broadcasted_iota(jnp.int32, cur.shape, 0)
    vals, idxs = [], []
    for _ in range(TOPK):
        m = jnp.max(cur, axis=0, keepdims=True)
        idx = jnp.min(jnp.where(cur == m, row, n_exp), axis=0, keepdims=True)
        vals.append(m)
        idxs.append(idx)
        cur = jnp.where(row == idx, -jnp.inf, cur)
    w = jnp.concatenate(vals, axis=0)
    e = jnp.exp(w - w[:1])
    w = e / jnp.sum(e, axis=0, keepdims=True)
    pid = pl.program_id(0)
    w_out[pl.ds(pid * lt.shape[1], lt.shape[1]), :] = w.T
    i_out[pl.ds(pid * lt.shape[1], lt.shape[1]), :] = jnp.concatenate(idxs, axis=0).T


@jax.jit
def kernel(x, W, router_scale):
    tokens, dim = x.shape
    n_exp = W.shape[0]
    grid = (tokens // BLOCK_M,)
    weights, indices = pl.pallas_call(
        _router_body,
        grid_spec=pltpu.PrefetchScalarGridSpec(
            num_scalar_prefetch=1,
            grid=grid,
            in_specs=[
                pl.BlockSpec((BLOCK_M, dim), lambda i, s: (i, 0)),
                pl.BlockSpec((n_exp, dim), lambda i, s: (0, 0)),
            ],
            out_specs=[
                pl.BlockSpec((16384, TOPK), lambda i, s: (0, 0)),
                pl.BlockSpec((16384, TOPK), lambda i, s: (0, 0)),
            ],
        ),
        out_shape=[
            jax.ShapeDtypeStruct((tokens, TOPK), jnp.float32),
            jax.ShapeDtypeStruct((tokens, TOPK), jnp.int32),
        ],
        compiler_params=pltpu.CompilerParams(
            dimension_semantics=("arbitrary",),
            vmem_limit_bytes=128 * 1024 * 1024,
        ),
    )(router_scale, x, W)
    return (weights, indices)


# speedup vs baseline: 1.1981x; 1.0018x over previous
import functools

import jax
import jax.numpy as jnp
from jax.experimental import pallas as pl
from jax.experimental.pallas import tpu as pltpu

TOPK = 8
BLOCK_M = 1024


def _router_body(scale_ref, x_ref, w_ref, w_out, i_out):
    lt = jax.lax.dot_general(
        w_ref[...], x_ref[...], (((1,), (1,)), ((), ())),
        preferred_element_type=jnp.float32)
    cur = lt * scale_ref[0]
    n_exp = cur.shape[0]
    row = jax.lax.broadcasted_iota(jnp.int32, cur.shape, 0)
    vals, idxs = [], []
    for _ in range(TOPK):
        m = jnp.max(cur, axis=0, keepdims=True)
        idx = jnp.min(jnp.where(cur == m, row, n_exp), axis=0, keepdims=True)
        vals.append(m)
        idxs.append(idx)
        cur = jnp.where(row == idx, -jnp.inf, cur)
    w = jnp.concatenate(vals, axis=0)
    e = jnp.exp(w - w[:1])
    w = e / jnp.sum(e, axis=0, keepdims=True)
    w_out[...] = w.T
    i_out[...] = jnp.concatenate(idxs, axis=0).T


@jax.jit
def kernel(x, W, router_scale):
    tokens, dim = x.shape
    n_exp = W.shape[0]
    grid = (tokens // BLOCK_M,)
    weights, indices = pl.pallas_call(
        _router_body,
        grid_spec=pltpu.PrefetchScalarGridSpec(
            num_scalar_prefetch=1,
            grid=grid,
            in_specs=[
                pl.BlockSpec((BLOCK_M, dim), lambda i, s: (i, 0)),
                pl.BlockSpec((n_exp, dim), lambda i, s: (0, 0)),
            ],
            out_specs=[
                pl.BlockSpec((BLOCK_M, TOPK), lambda i, s: (i, 0)),
                pl.BlockSpec((BLOCK_M, TOPK), lambda i, s: (i, 0)),
            ],
        ),
        out_shape=[
            jax.ShapeDtypeStruct((tokens, TOPK), jnp.float32),
            jax.ShapeDtypeStruct((tokens, TOPK), jnp.int32),
        ],
        compiler_params=pltpu.CompilerParams(
            dimension_semantics=("parallel",),
            vmem_limit_bytes=128 * 1024 * 1024,
        ),
    )(router_scale, x, W)
    return (weights, indices)
